# baseline (speedup 1.0000x reference)
import functools
import math

import jax
import jax.numpy as jnp
from jax import lax
from jax.experimental import pallas as pl
from jax.experimental.pallas import tpu as pltpu

_TAU = 0.1
_UNIFORM_WEIGHT = 0.1
_TOPK = 5
_TILE = 512
_LOG2E = 1.4426950408889634
_ZSCALE = math.sqrt(10.0 * _LOG2E)


def _prep_kernel(z_ref, s_ref, znc_ref, sbb_ref, sqsp_ref):
    S = s_ref[...]
    sqsp_ref[...] = jnp.sum(S * S, axis=1, keepdims=True) + 512.0
    sbb_ref[...] = S.astype(jnp.bfloat16)
    Z = z_ref[...]
    nsq = jnp.sum(Z * Z, axis=1, keepdims=True)
    r = jnp.float32(_ZSCALE) / jnp.maximum(jnp.sqrt(nsq), 1e-12)
    znc_ref[...] = (Z * r).astype(jnp.bfloat16)


def _loss_kernel(znc_ref, sbb_ref, sqsr_ref, choice_ref, info_ref, unif_ref,
                 *, k, tile):
    i = pl.program_id(0)
    B = znc_ref.shape[0]
    row0 = i * tile

    sm2 = sbb_ref[pl.ds(row0, tile), :] * jnp.bfloat16(-2.0)
    G2 = lax.dot_general(sm2, sbb_ref[...], (((1,), (1,)), ((), ())),
                         preferred_element_type=jnp.float32)
    zn_i = znc_ref[pl.ds(row0, tile), :]
    simp = lax.dot_general(zn_i, znc_ref[...], (((1,), (1,)), ((), ())),
                           preferred_element_type=jnp.float32)

    v = G2 + sqsr_ref[...]
    col = lax.broadcasted_iota(jnp.int32, (tile, B), 1)
    row_l = lax.broadcasted_iota(jnp.int32, (tile, 1), 0) + row0
    inf = jnp.float32(jnp.inf)
    v = jnp.where(col == row_l, jnp.float32(3.0e38), v)
    ki = (lax.bitcast_convert_type(v, jnp.int32) & jnp.int32(~0xFFF)) | col
    key = lax.bitcast_convert_type(ki, jnp.float32)

    choice = choice_ref[0]
    pos = jnp.zeros((tile, 1), jnp.int32)
    mkey = jnp.min(key, axis=1, keepdims=True)
    for rnd in range(k):
        idx = lax.bitcast_convert_type(mkey, jnp.int32) & jnp.int32(0xFFF)
        pos = jnp.where(choice == rnd, idx, pos)
        if rnd + 1 < k:
            mkey = jnp.min(jnp.where(key > mkey, key, inf),
                           axis=1, keepdims=True)

    e = jnp.exp2(simp)
    u = jnp.exp2(simp * jnp.float32(0.4))
    denom = jnp.sum(e, axis=1, keepdims=True)
    numer = jnp.sum(jnp.where(col == pos, e, 0.0), axis=1, keepdims=True)
    info = jnp.sum(-jnp.log(numer / (denom + 1e-8)))
    usum = jnp.sum(u)

    zr = lax.broadcasted_iota(jnp.int32, (8, 128), 0) \
        + lax.broadcasted_iota(jnp.int32, (8, 128), 1)
    first = zr == 0
    info_ref[...] = jnp.where(first, info, 0.0)
    unif_ref[...] = jnp.where(first, usum, 0.0)


def kernel(z, binding_scores):
    B, D = z.shape
    F = binding_scores.shape[1]
    k = min(_TOPK, B - 1)
    tile = _TILE if B % _TILE == 0 else B
    nsteps = B // tile
    choice = jax.random.randint(jax.random.key(12345), (B,), 0, k)
    choice3 = choice.astype(jnp.int32).reshape(nsteps, tile, 1)

    par = pltpu.CompilerParams(dimension_semantics=("parallel",))
    znc, sbb, sqsp = pl.pallas_call(
        _prep_kernel,
        grid=(nsteps,),
        in_specs=[pl.BlockSpec((tile, D), lambda i: (i, 0)),
                  pl.BlockSpec((tile, F), lambda i: (i, 0))],
        out_specs=[pl.BlockSpec((tile, D), lambda i: (i, 0)),
                   pl.BlockSpec((tile, F), lambda i: (i, 0)),
                   pl.BlockSpec((tile, 1), lambda i: (i, 0))],
        out_shape=[jax.ShapeDtypeStruct((B, D), jnp.bfloat16),
                   jax.ShapeDtypeStruct((B, F), jnp.bfloat16),
                   jax.ShapeDtypeStruct((B, 1), jnp.float32)],
        compiler_params=par,
    )(z, binding_scores)
    sqsr = sqsp.reshape(1, B)

    body = functools.partial(_loss_kernel, k=k, tile=tile)
    info_part, unif_part = pl.pallas_call(
        body,
        grid=(nsteps,),
        in_specs=[
            pl.BlockSpec((B, D), lambda i: (0, 0)),
            pl.BlockSpec((B, F), lambda i: (0, 0)),
            pl.BlockSpec((1, B), lambda i: (0, 0)),
            pl.BlockSpec((1, tile, 1), lambda i: (i, 0, 0)),
        ],
        out_specs=[pl.BlockSpec((8, 128), lambda i: (i, 0)),
                   pl.BlockSpec((8, 128), lambda i: (i, 0))],
        out_shape=[jax.ShapeDtypeStruct((nsteps * 8, 128), jnp.float32),
                   jax.ShapeDtypeStruct((nsteps * 8, 128), jnp.float32)],
        compiler_params=par,
    )(znc, sbb, sqsr, choice3)

    L_info = jnp.sum(info_part) / B
    L_unif = jnp.log(jnp.sum(unif_part) * jnp.exp(-4.0) / (B * B) + 1e-8)
    return L_info + _UNIFORM_WEIGHT * L_unif


# speedup vs baseline: 1.1094x; 1.1094x over previous
import functools
import math

import jax
import jax.numpy as jnp
from jax import lax
from jax.experimental import pallas as pl
from jax.experimental.pallas import tpu as pltpu

_TAU = 0.1
_UNIFORM_WEIGHT = 0.1
_TOPK = 5
_TILE = 512
_LOG2E = 1.4426950408889634
_ZSCALE = math.sqrt(10.0 * _LOG2E)


def _loss_kernel(z_ref, s_ref, choice_ref, info_ref, unif_ref,
                 znc_ref, sbb_ref, sqsr_ref, *, k, tile):
    i = pl.program_id(0)
    B = z_ref.shape[0]
    row0 = i * tile

    @pl.when(i == 0)
    def _():
        S = s_ref[...]
        sqs = jnp.sum(S * S, axis=1, keepdims=True)
        sqsr_ref[...] = sqs.T + 512.0
        sbb_ref[...] = S.astype(jnp.bfloat16)
        Z = z_ref[...]
        nsq = jnp.sum(Z * Z, axis=1, keepdims=True)
        r = jnp.float32(_ZSCALE) / jnp.maximum(jnp.sqrt(nsq), 1e-12)
        znc_ref[...] = (Z * r).astype(jnp.bfloat16)
        info_ref[...] = jnp.zeros((1, 1), jnp.float32)
        unif_ref[...] = jnp.zeros((1, 1), jnp.float32)

    sm2 = sbb_ref[pl.ds(row0, tile), :] * jnp.bfloat16(-2.0)
    G2 = lax.dot_general(sm2, sbb_ref[...], (((1,), (1,)), ((), ())),
                         preferred_element_type=jnp.float32)
    zn_i = znc_ref[pl.ds(row0, tile), :]
    simp = lax.dot_general(zn_i, znc_ref[...], (((1,), (1,)), ((), ())),
                           preferred_element_type=jnp.float32)

    v = G2 + sqsr_ref[...]
    col = lax.broadcasted_iota(jnp.int32, (tile, B), 1)
    row_l = lax.broadcasted_iota(jnp.int32, (tile, 1), 0) + row0
    inf = jnp.float32(jnp.inf)
    v = jnp.where(col == row_l, jnp.float32(3.0e38), v)
    ki = (lax.bitcast_convert_type(v, jnp.int32) & jnp.int32(~0xFFF)) | col
    key = lax.bitcast_convert_type(ki, jnp.float32)

    choice = choice_ref[0]
    pos = jnp.zeros((tile, 1), jnp.int32)
    mkey = jnp.min(key, axis=1, keepdims=True)
    for rnd in range(k):
        idx = lax.bitcast_convert_type(mkey, jnp.int32) & jnp.int32(0xFFF)
        pos = jnp.where(choice == rnd, idx, pos)
        if rnd + 1 < k:
            mkey = jnp.min(jnp.where(key > mkey, key, inf),
                           axis=1, keepdims=True)

    e = jnp.exp2(simp)
    u = jnp.exp2(simp * jnp.float32(0.4))
    denom = jnp.sum(e, axis=1, keepdims=True)
    numer = jnp.sum(jnp.where(col == pos, e, 0.0), axis=1, keepdims=True)
    info = jnp.sum(-jnp.log(numer / (denom + 1e-8)), keepdims=True)
    usum = jnp.sum(u, keepdims=True)

    info_ref[...] += info.reshape(1, 1)
    unif_ref[...] += usum.reshape(1, 1)


def kernel(z, binding_scores):
    B, D = z.shape
    F = binding_scores.shape[1]
    k = min(_TOPK, B - 1)
    tile = _TILE if B % _TILE == 0 else B
    nsteps = B // tile
    choice = jax.random.randint(jax.random.key(12345), (B,), 0, k)
    choice3 = choice.astype(jnp.int32).reshape(nsteps, tile, 1)
    body = functools.partial(_loss_kernel, k=k, tile=tile)
    info_sum, unif_sum = pl.pallas_call(
        body,
        grid=(nsteps,),
        in_specs=[
            pl.BlockSpec((B, D), lambda i: (0, 0)),
            pl.BlockSpec((B, F), lambda i: (0, 0)),
            pl.BlockSpec((1, tile, 1), lambda i: (i, 0, 0)),
        ],
        out_specs=[pl.BlockSpec((1, 1), lambda i: (0, 0)),
                   pl.BlockSpec((1, 1), lambda i: (0, 0))],
        out_shape=[jax.ShapeDtypeStruct((1, 1), jnp.float32),
                   jax.ShapeDtypeStruct((1, 1), jnp.float32)],
        scratch_shapes=[
            pltpu.VMEM((B, D), jnp.bfloat16),
            pltpu.VMEM((B, F), jnp.bfloat16),
            pltpu.VMEM((1, B), jnp.float32),
        ],
    )(z, binding_scores, choice3)
    L_info = info_sum[0, 0] / B
    L_unif = jnp.log(unif_sum[0, 0] * jnp.exp(-4.0) / (B * B) + 1e-8)
    return L_info + _UNIFORM_WEIGHT * L_unif


# speedup vs baseline: 1.1399x; 1.0275x over previous
import functools
import math

import jax
import jax.numpy as jnp
from jax import lax
from jax.experimental import pallas as pl
from jax.experimental.pallas import tpu as pltpu

_TAU = 0.1
_UNIFORM_WEIGHT = 0.1
_TOPK = 5
_TILE = 1024
_LOG2E = 1.4426950408889634
_ZSCALE = math.sqrt(10.0 * _LOG2E)


def _loss_kernel(z_ref, s_ref, choice_ref, info_ref, unif_ref,
                 znc_ref, sbb_ref, sqsr_ref, *, k, tile):
    i = pl.program_id(0)
    B = z_ref.shape[0]
    row0 = i * tile

    @pl.when(i == 0)
    def _():
        S = s_ref[...]
        sqs = jnp.sum(S * S, axis=1, keepdims=True)
        sqsr_ref[...] = sqs.T + 512.0
        sbb_ref[...] = S.astype(jnp.bfloat16)
        Z = z_ref[...]
        nsq = jnp.sum(Z * Z, axis=1, keepdims=True)
        r = jnp.float32(_ZSCALE) / jnp.maximum(jnp.sqrt(nsq), 1e-12)
        znc_ref[...] = (Z * r).astype(jnp.bfloat16)
        info_ref[...] = jnp.zeros((1, 1), jnp.float32)
        unif_ref[...] = jnp.zeros((1, 1), jnp.float32)

    sm2 = sbb_ref[pl.ds(row0, tile), :] * jnp.bfloat16(-2.0)
    G2 = lax.dot_general(sm2, sbb_ref[...], (((1,), (1,)), ((), ())),
                         preferred_element_type=jnp.float32)
    zn_i = znc_ref[pl.ds(row0, tile), :]
    simp = lax.dot_general(zn_i, znc_ref[...], (((1,), (1,)), ((), ())),
                           preferred_element_type=jnp.float32)

    v = G2 + sqsr_ref[...]
    col = lax.broadcasted_iota(jnp.int32, (tile, B), 1)
    row_l = lax.broadcasted_iota(jnp.int32, (tile, 1), 0) + row0
    inf = jnp.float32(jnp.inf)
    v = jnp.where(col == row_l, jnp.float32(3.0e38), v)
    ki = (lax.bitcast_convert_type(v, jnp.int32) & jnp.int32(~0xFFF)) | col
    key = lax.bitcast_convert_type(ki, jnp.float32)

    choice = choice_ref[0]
    pos = jnp.zeros((tile, 1), jnp.int32)
    mkey = jnp.min(key, axis=1, keepdims=True)
    for rnd in range(k):
        idx = lax.bitcast_convert_type(mkey, jnp.int32) & jnp.int32(0xFFF)
        pos = jnp.where(choice == rnd, idx, pos)
        if rnd + 1 < k:
            mkey = jnp.min(jnp.where(key > mkey, key, inf),
                           axis=1, keepdims=True)

    e = jnp.exp2(simp)
    u = jnp.exp2(simp * jnp.float32(0.4))
    denom = jnp.sum(e, axis=1, keepdims=True)
    numer = jnp.sum(jnp.where(col == pos, e, 0.0), axis=1, keepdims=True)
    info = jnp.sum(-jnp.log(numer / (denom + 1e-8)), keepdims=True)
    usum = jnp.sum(u, keepdims=True)

    info_ref[...] += info.reshape(1, 1)
    unif_ref[...] += usum.reshape(1, 1)


def kernel(z, binding_scores):
    B, D = z.shape
    F = binding_scores.shape[1]
    k = min(_TOPK, B - 1)
    tile = _TILE if B % _TILE == 0 else B
    nsteps = B // tile
    choice = jax.random.randint(jax.random.key(12345), (B,), 0, k)
    choice3 = choice.astype(jnp.int32).reshape(nsteps, tile, 1)
    body = functools.partial(_loss_kernel, k=k, tile=tile)
    info_sum, unif_sum = pl.pallas_call(
        body,
        grid=(nsteps,),
        in_specs=[
            pl.BlockSpec((B, D), lambda i: (0, 0)),
            pl.BlockSpec((B, F), lambda i: (0, 0)),
            pl.BlockSpec((1, tile, 1), lambda i: (i, 0, 0)),
        ],
        out_specs=[pl.BlockSpec((1, 1), lambda i: (0, 0)),
                   pl.BlockSpec((1, 1), lambda i: (0, 0))],
        out_shape=[jax.ShapeDtypeStruct((1, 1), jnp.float32),
                   jax.ShapeDtypeStruct((1, 1), jnp.float32)],
        scratch_shapes=[
            pltpu.VMEM((B, D), jnp.bfloat16),
            pltpu.VMEM((B, F), jnp.bfloat16),
            pltpu.VMEM((1, B), jnp.float32),
        ],
        compiler_params=pltpu.CompilerParams(
            vmem_limit_bytes=100 * 1024 * 1024),
    )(z, binding_scores, choice3)
    L_info = info_sum[0, 0] / B
    L_unif = jnp.log(unif_sum[0, 0] * jnp.exp(-4.0) / (B * B) + 1e-8)
    return L_info + _UNIFORM_WEIGHT * L_unif


# speedup vs baseline: 1.1608x; 1.0183x over previous
import functools
import math

import jax
import jax.numpy as jnp
from jax import lax
from jax.experimental import pallas as pl
from jax.experimental.pallas import tpu as pltpu

_TAU = 0.1
_UNIFORM_WEIGHT = 0.1
_TOPK = 5
_TILE = 1024
_LOG2E = 1.4426950408889634
_ZSCALE = math.sqrt(10.0 * _LOG2E)


def _loss_kernel(z_ref, s_ref, choice_ref, info_ref, unif_ref,
                 znc_ref, sbl_ref, sbr_ref, *, k, tile):
    i = pl.program_id(0)
    B = z_ref.shape[0]
    row0 = i * tile

    F = s_ref.shape[1]

    @pl.when(i == 0)
    def _():
        S = s_ref[...]
        sqs = jnp.sum(S * S, axis=1, keepdims=True) + 512.0
        ones = jnp.ones((B, 1), jnp.float32)
        pad0 = jnp.zeros((B, 126), jnp.float32)
        hi = sqs.astype(jnp.bfloat16).astype(jnp.float32)
        lo = sqs - hi
        sbl_ref[...] = jnp.concatenate(
            [-2.0 * S, ones, ones, pad0], axis=1).astype(jnp.bfloat16)
        sbr_ref[...] = jnp.concatenate(
            [S, hi, lo, pad0], axis=1).astype(jnp.bfloat16)
        Z = z_ref[...]
        nsq = jnp.sum(Z * Z, axis=1, keepdims=True)
        r = jnp.float32(_ZSCALE) / jnp.maximum(jnp.sqrt(nsq), 1e-12)
        znc_ref[...] = (Z * r).astype(jnp.bfloat16)
        info_ref[...] = jnp.zeros((1, 1), jnp.float32)
        unif_ref[...] = jnp.zeros((1, 1), jnp.float32)

    sl_i = sbl_ref[pl.ds(row0, tile), :]
    v = lax.dot_general(sl_i, sbr_ref[...], (((1,), (1,)), ((), ())),
                        preferred_element_type=jnp.float32)
    zn_i = znc_ref[pl.ds(row0, tile), :]
    simp = lax.dot_general(zn_i, znc_ref[...], (((1,), (1,)), ((), ())),
                           preferred_element_type=jnp.float32)

    col = lax.broadcasted_iota(jnp.int32, (tile, B), 1)
    row_l = lax.broadcasted_iota(jnp.int32, (tile, 1), 0) + row0
    inf = jnp.float32(jnp.inf)
    v = jnp.where(col == row_l, jnp.float32(3.0e38), v)
    ki = (lax.bitcast_convert_type(v, jnp.int32) & jnp.int32(~0xFFF)) | col
    key = lax.bitcast_convert_type(ki, jnp.float32)

    choice = choice_ref[0]
    pos = jnp.zeros((tile, 1), jnp.int32)
    mkey = jnp.min(key, axis=1, keepdims=True)
    for rnd in range(k):
        idx = lax.bitcast_convert_type(mkey, jnp.int32) & jnp.int32(0xFFF)
        pos = jnp.where(choice == rnd, idx, pos)
        if rnd + 1 < k:
            mkey = jnp.min(jnp.where(key > mkey, key, inf),
                           axis=1, keepdims=True)

    e = jnp.exp2(simp)
    u = jnp.exp2(simp * jnp.float32(0.4))
    denom = jnp.sum(e, axis=1, keepdims=True)
    numer = jnp.sum(jnp.where(col == pos, e, 0.0), axis=1, keepdims=True)
    info = jnp.sum(-jnp.log(numer / (denom + 1e-8)), keepdims=True)
    usum = jnp.sum(u, keepdims=True)

    info_ref[...] += info.reshape(1, 1)
    unif_ref[...] += usum.reshape(1, 1)


def kernel(z, binding_scores):
    B, D = z.shape
    F = binding_scores.shape[1]
    k = min(_TOPK, B - 1)
    tile = _TILE if B % _TILE == 0 else B
    nsteps = B // tile
    choice = jax.random.randint(jax.random.key(12345), (B,), 0, k)
    choice3 = choice.astype(jnp.int32).reshape(nsteps, tile, 1)
    body = functools.partial(_loss_kernel, k=k, tile=tile)
    info_sum, unif_sum = pl.pallas_call(
        body,
        grid=(nsteps,),
        in_specs=[
            pl.BlockSpec((B, D), lambda i: (0, 0)),
            pl.BlockSpec((B, F), lambda i: (0, 0)),
            pl.BlockSpec((1, tile, 1), lambda i: (i, 0, 0)),
        ],
        out_specs=[pl.BlockSpec((1, 1), lambda i: (0, 0)),
                   pl.BlockSpec((1, 1), lambda i: (0, 0))],
        out_shape=[jax.ShapeDtypeStruct((1, 1), jnp.float32),
                   jax.ShapeDtypeStruct((1, 1), jnp.float32)],
        scratch_shapes=[
            pltpu.VMEM((B, D), jnp.bfloat16),
            pltpu.VMEM((B, F + 128), jnp.bfloat16),
            pltpu.VMEM((B, F + 128), jnp.bfloat16),
        ],
        compiler_params=pltpu.CompilerParams(
            vmem_limit_bytes=100 * 1024 * 1024),
    )(z, binding_scores, choice3)
    L_info = info_sum[0, 0] / B
    L_unif = jnp.log(unif_sum[0, 0] * jnp.exp(-4.0) / (B * B) + 1e-8)
    return L_info + _UNIFORM_WEIGHT * L_unif
